# baseline (speedup 1.0000x reference)
import functools
import jax
import jax.numpy as jnp
from jax import lax
from jax.experimental import pallas as pl
from jax.experimental.pallas import tpu as pltpu
from jax.experimental.pallas import tpu_sc as plsc

N_ROWS = 100000
D_IN = 512
D_OUT = 256
NC = 2
NS = 16
NW = NC * NS
CHUNK = 40
NCHUNK_TOT = N_ROWS // CHUNK
ITERS = -(-NCHUNK_TOT // NW)
NBI = 3
NBO = 2


def _sc_body(w_hbm, out_hbm, in_buf, out_buf, in_sem0, in_sem1, in_sem2,
             out_sem0, out_sem1):
    wid = lax.axis_index("s") * NC + lax.axis_index("c")
    iota2 = lax.iota(jnp.int32, 16) * 2
    in_sems = (in_sem0, in_sem1, in_sem2)
    out_sems = (out_sem0, out_sem1)

    def chunk_of(t):
        c = wid + t * NW
        return jnp.where(c < NCHUNK_TOT, c, wid)

    def start_in(t, slot):
        r0 = pl.multiple_of(chunk_of(t) * CHUNK, 8)
        pltpu.make_async_copy(
            w_hbm.at[pl.ds(r0, CHUNK)], in_buf.at[slot], in_sems[slot]
        ).start()

    def wait_in(slot):
        pltpu.make_async_copy(
            w_hbm.at[pl.ds(0, CHUNK)], in_buf.at[slot], in_sems[slot]
        ).wait()

    def start_out(t, slot):
        r0 = pl.multiple_of(chunk_of(t) * CHUNK, 8)
        pltpu.make_async_copy(
            out_buf.at[slot], out_hbm.at[pl.ds(r0, CHUNK)], out_sems[slot]
        ).start()

    def wait_out(slot):
        pltpu.make_async_copy(
            out_buf.at[slot], out_hbm.at[pl.ds(0, CHUNK)], out_sems[slot]
        ).wait()

    def compute(islot, oslot):
        def row_body(r):
            rvec = jnp.full((16,), r, dtype=jnp.int32)
            for v in range(D_OUT // 16):
                ce = iota2 + (32 * v)
                e = plsc.load_gather(in_buf.at[islot], [rvec, ce])
                o = plsc.load_gather(in_buf.at[islot], [rvec, ce + 1])
                out_buf[oslot, r, pl.ds(16 * v, 16)] = (e + o) * 0.5

        plsc.parallel_loop(0, CHUNK, unroll=2)(row_body)

    for s in range(NBI):
        start_in(s, s)

    def group_body(g, carry):
        t0 = 6 * g
        for j in range(6):
            t = t0 + j
            isl = j % NBI
            osl = j % NBO

            @pl.when(t >= NBO)
            def _():
                wait_out(osl)

            wait_in(isl)
            compute(isl, osl)
            start_out(t, osl)

            @pl.when(t + NBI < ITERS)
            def _():
                start_in(t + NBI, isl)

        return carry

    NGROUPS = (ITERS - 1) // 6
    lax.fori_loop(0, NGROUPS, group_body, 0)

    t = ITERS - 1
    wait_out(t % NBO)
    wait_in(t % NBI)
    compute(t % NBI, t % NBO)
    start_out(t, t % NBO)
    wait_out(0)
    wait_out(1)


def kernel(W):
    mesh = plsc.VectorSubcoreMesh(core_axis_name="c", subcore_axis_name="s")
    f = functools.partial(
        pl.kernel,
        mesh=mesh,
        out_type=jax.ShapeDtypeStruct((N_ROWS, D_OUT), jnp.float32),
        compiler_params=pltpu.CompilerParams(needs_layout_passes=False),
        scratch_types=[
            pltpu.VMEM((NBI, CHUNK, D_IN), jnp.float32),
            pltpu.VMEM((NBO, CHUNK, D_OUT), jnp.float32),
            pltpu.SemaphoreType.DMA,
            pltpu.SemaphoreType.DMA,
            pltpu.SemaphoreType.DMA,
            pltpu.SemaphoreType.DMA,
            pltpu.SemaphoreType.DMA,
        ],
    )(_sc_body)
    return f(W)
